# Initial kernel scaffold; baseline (speedup 1.0000x reference)
import jax
import jax.numpy as jnp
from jax.experimental import pallas as pl


def kernel(hidden_states, probs, routing_map):
    raise NotImplementedError("write your pallas kernel here")


# speedup vs baseline: 1.5853x; 1.5853x over previous
import functools

import jax
import jax.numpy as jnp
from jax import lax
from jax.experimental import pallas as pl
from jax.experimental.pallas import tpu as pltpu
from jax.experimental.pallas import tpu_sc as plsc

T = 16384
E = 8
D = 4096
TOPK = 2
FLAT = T * E
OUT = T * TOPK
NW = 32
CHUNK = FLAT // NW
ROWS_PW = OUT // NW
L = 16
RG = 8

_MESH = dict(core_axis_name="c", subcore_axis_name="s")

_DNUMS = lax.GatherDimensionNumbers(
    offset_dims=(), collapsed_slice_dims=(0,), start_index_map=(0,))


def _wid():
    return lax.axis_index("s") * 2 + lax.axis_index("c")


def _dyn_gather(x, idx):
    return lax.gather(x, idx[:, None], dimension_numbers=_DNUMS,
                      slice_sizes=(1,),
                      mode=lax.GatherScatterMode.PROMISE_IN_BOUNDS)


def _iota():
    return lax.iota(jnp.int32, L)


def _shift_cumsum(x):
    iota = _iota()
    y = x
    for s in (1, 2, 4, 8):
        sh = _dyn_gather(y, jnp.maximum(iota - s, 0))
        y = y + jnp.where(iota >= s, sh, 0)
    return y


def _bcast_lane(y, lane):
    return _dyn_gather(y, jnp.full((L,), lane, jnp.int32))


def _sum_scalar(v):
    return _shift_cumsum(v)[L - 1]


@functools.partial(
    pl.kernel,
    out_type=jax.ShapeDtypeStruct((NW * 8,), jnp.int32),
    mesh=plsc.VectorSubcoreMesh(**_MESH),
    compiler_params=pltpu.CompilerParams(needs_layout_passes=False),
    scratch_types=[
        pltpu.VMEM((CHUNK,), jnp.int32),
        pltpu.VMEM((L,), jnp.int32),
    ],
)
def _count_kernel(maskf_hbm, counts_hbm, chunk_v, cnt_v):
    w = _wid()
    pltpu.sync_copy(maskf_hbm.at[pl.ds(w * CHUNK, CHUNK)], chunk_v)

    def body(i, acc):
        return acc + chunk_v[pl.ds(i * L, L)]

    acc = lax.fori_loop(0, CHUNK // L, body, jnp.zeros((L,), jnp.int32))
    total = _sum_scalar(acc)
    cnt_v[...] = jnp.full((L,), total, jnp.int32)
    pltpu.sync_copy(cnt_v.at[pl.ds(0, 8)], counts_hbm.at[pl.ds(w * 8, 8)])


@functools.partial(
    pl.kernel,
    out_type=(
        jax.ShapeDtypeStruct((OUT,), jnp.int32),
        jax.ShapeDtypeStruct((E,), jnp.int32),
    ),
    mesh=plsc.VectorSubcoreMesh(**_MESH),
    compiler_params=pltpu.CompilerParams(needs_layout_passes=False),
    scratch_types=[
        pltpu.VMEM((CHUNK,), jnp.int32),
        pltpu.VMEM((NW * 8,), jnp.int32),
        pltpu.VMEM((CHUNK,), jnp.int32),
        pltpu.VMEM((128,), jnp.int32),
        pltpu.VMEM((128,), jnp.int32),
        pltpu.VMEM((L,), jnp.int32),
    ],
)
def _compact_kernel(maskf_hbm, counts_hbm, sel_hbm, tpe_hbm,
                    chunk_v, cnt_v, sel_buf, idxg, valg, tpe_v):
    w = _wid()
    iota = _iota()
    pltpu.sync_copy(counts_hbm, cnt_v)
    c0 = plsc.load_gather(cnt_v, [iota * 8])
    c1 = plsc.load_gather(cnt_v, [(iota + L) * 8])
    pv = jnp.where(iota < w, c0, 0) + jnp.where(iota + L < w, c1, 0)
    prefix = _sum_scalar(pv)

    @pl.when(w == 0)
    def _():
        acc = jnp.zeros((L,), jnp.int32)
        for e in range(E):
            src = c0 if e < 4 else c1
            s = _sum_scalar(jnp.where(iota // 4 == (e % 4), src, 0))
            acc = acc + s * jnp.where(iota == e, 1, 0)
        tpe_v[...] = acc
        pltpu.sync_copy(tpe_v.at[pl.ds(0, E)], tpe_hbm)

    pltpu.sync_copy(maskf_hbm.at[pl.ds(w * CHUNK, CHUNK)], chunk_v)
    base_flat = w * CHUNK

    def comp(g, off_vec):
        m = chunk_v[pl.ds(g * L, L)]
        incl = _shift_cumsum(m)
        dst = jnp.maximum(off_vec + incl - 1, 0)
        plsc.store_scatter(sel_buf, [dst], base_flat + g * L + iota,
                           mask=m == 1)
        return off_vec + _bcast_lane(incl, L - 1)

    off_vec = lax.fori_loop(0, CHUNK // L, comp, jnp.zeros((L,), jnp.int32))
    count = off_vec[0]

    def scat(j, _):
        base = j * 128
        for k in range(8):
            pos = jnp.minimum(base + k * L + iota, count - 1)
            valg[pl.ds(k * L, L)] = plsc.load_gather(sel_buf, [pos])
            idxg[pl.ds(k * L, L)] = prefix + pos
        pltpu.sync_copy(valg, sel_hbm.at[idxg])
        return 0

    lax.fori_loop(0, (count + 127) // 128, scat, 0)


@functools.partial(
    pl.kernel,
    out_type=(
        jax.ShapeDtypeStruct((OUT, D), jnp.float32),
        jax.ShapeDtypeStruct((OUT,), jnp.float32),
    ),
    mesh=plsc.VectorSubcoreMesh(**_MESH),
    compiler_params=pltpu.CompilerParams(needs_layout_passes=False),
    scratch_types=[
        pltpu.VMEM((ROWS_PW,), jnp.int32),
        pltpu.VMEM((ROWS_PW,), jnp.int32),
        pltpu.VMEM((ROWS_PW,), jnp.float32),
        pltpu.VMEM((RG, D), jnp.float32),
        pltpu.VMEM((RG, D), jnp.float32),
        pltpu.SemaphoreType.DMA,
        pltpu.SemaphoreType.DMA,
        pltpu.SemaphoreType.DMA,
    ],
)
def _gather_kernel(hidden_hbm, probsf_hbm, sel_hbm, out_hbm, pprobs_hbm,
                   sel_v, tok_v, pr_v, rbuf0, rbuf1, sem0, sem1, psem):
    w = _wid()
    row0 = w * ROWS_PW
    pltpu.sync_copy(sel_hbm.at[pl.ds(row0, ROWS_PW)], sel_v)

    def tok(i, _):
        tok_v[pl.ds(i * L, L)] = lax.bitwise_and(sel_v[pl.ds(i * L, L)], T - 1)
        return 0

    lax.fori_loop(0, ROWS_PW // L, tok, 0)

    for k in range(ROWS_PW // 128):
        pltpu.async_copy(probsf_hbm.at[sel_v.at[pl.ds(k * 128, 128)]],
                         pr_v.at[pl.ds(k * 128, 128)], psem)
    for k in range(ROWS_PW // 128):
        pltpu.make_async_copy(probsf_hbm.at[sel_v.at[pl.ds(k * 128, 128)]],
                              pr_v.at[pl.ds(k * 128, 128)], psem).wait()
    pltpu.sync_copy(pr_v, pprobs_hbm.at[pl.ds(row0, ROWS_PW)])

    ng = ROWS_PW // RG

    def start(g, buf, sem):
        pltpu.async_copy(hidden_hbm.at[tok_v.at[pl.ds(g * RG, RG)]], buf, sem)

    def wait(buf, sem):
        pltpu.make_async_copy(hidden_hbm.at[tok_v.at[pl.ds(0, RG)]], buf,
                              sem).wait()

    start(0, rbuf0, sem0)

    def rows(i, _):
        g0 = i * 2
        wait(rbuf0, sem0)
        start(g0 + 1, rbuf1, sem1)
        pltpu.sync_copy(rbuf0, out_hbm.at[pl.ds(row0 + g0 * RG, RG)])
        wait(rbuf1, sem1)

        @pl.when(g0 + 2 < ng)
        def _():
            start(g0 + 2, rbuf0, sem0)

        pltpu.sync_copy(rbuf1, out_hbm.at[pl.ds(row0 + (g0 + 1) * RG, RG)])
        return 0

    lax.fori_loop(0, ng // 2, rows, 0)


def kernel(hidden_states, probs, routing_map):
    maskf = routing_map.T.astype(jnp.int32).reshape(-1)
    probsf = probs.T.reshape(-1)
    counts = _count_kernel(maskf)
    sel, tokens_per_expert = _compact_kernel(maskf, counts)
    permuted_tokens, permuted_probs = _gather_kernel(hidden_states, probsf, sel)
    return permuted_tokens, tokens_per_expert, permuted_probs


# speedup vs baseline: 1.9942x; 1.2580x over previous
import functools

import jax
import jax.numpy as jnp
from jax import lax
from jax.experimental import pallas as pl
from jax.experimental.pallas import tpu as pltpu
from jax.experimental.pallas import tpu_sc as plsc

T = 16384
E = 8
D = 4096
TOPK = 2
FLAT = T * E
OUT = T * TOPK
NW = 32
CHUNK = FLAT // NW
ROWS_PW = OUT // NW
L = 16
RG = 8

_MESH = dict(core_axis_name="c", subcore_axis_name="s")

_DNUMS = lax.GatherDimensionNumbers(
    offset_dims=(), collapsed_slice_dims=(0,), start_index_map=(0,))


def _wid():
    return lax.axis_index("s") * 2 + lax.axis_index("c")


def _dyn_gather(x, idx):
    return lax.gather(x, idx[:, None], dimension_numbers=_DNUMS,
                      slice_sizes=(1,),
                      mode=lax.GatherScatterMode.PROMISE_IN_BOUNDS)


def _iota():
    return lax.iota(jnp.int32, L)


def _shift_cumsum(x):
    iota = _iota()
    y = x
    for s in (1, 2, 4, 8):
        sh = _dyn_gather(y, jnp.maximum(iota - s, 0))
        y = y + jnp.where(iota >= s, sh, 0)
    return y


def _sum_scalar(v):
    return _shift_cumsum(v)[L - 1]


def _splat(x):
    return jnp.full((L,), x, jnp.int32)


@functools.partial(
    pl.kernel,
    out_type=jax.ShapeDtypeStruct((NW * 8,), jnp.int32),
    mesh=plsc.VectorSubcoreMesh(**_MESH),
    compiler_params=pltpu.CompilerParams(needs_layout_passes=False),
    scratch_types=[
        pltpu.VMEM((CHUNK,), jnp.int32),
        pltpu.VMEM((L,), jnp.int32),
    ],
)
def _count_kernel(maskf_hbm, counts_hbm, chunk_v, cnt_v):
    w = _wid()
    pltpu.sync_copy(maskf_hbm.at[pl.ds(w * CHUNK, CHUNK)], chunk_v)

    def body(i, acc):
        return acc + chunk_v[pl.ds(i * L, L)]

    acc = lax.fori_loop(0, CHUNK // L, body, jnp.zeros((L,), jnp.int32),
                        unroll=8)
    total = _sum_scalar(acc)
    cnt_v[...] = _splat(total)
    pltpu.sync_copy(cnt_v.at[pl.ds(0, 8)], counts_hbm.at[pl.ds(w * 8, 8)])


@functools.partial(
    pl.kernel,
    out_type=(
        jax.ShapeDtypeStruct((FLAT,), jnp.int32),
        jax.ShapeDtypeStruct((FLAT,), jnp.float32),
        jax.ShapeDtypeStruct((E,), jnp.int32),
    ),
    mesh=plsc.VectorSubcoreMesh(**_MESH),
    compiler_params=pltpu.CompilerParams(needs_layout_passes=False),
    scratch_types=[
        pltpu.VMEM((CHUNK,), jnp.int32),
        pltpu.VMEM((CHUNK,), jnp.float32),
        pltpu.VMEM((NW * 8,), jnp.int32),
        pltpu.VMEM((CHUNK,), jnp.int32),
        pltpu.VMEM((CHUNK,), jnp.float32),
        pltpu.VMEM((L,), jnp.int32),
    ],
)
def _compact_kernel(maskf_hbm, probsf_hbm, counts_hbm,
                    selpad_hbm, prpad_hbm, tpe_hbm,
                    chunk_v, pchunk_v, cnt_v, sel_buf, pr_buf, tpe_v):
    w = _wid()
    iota = _iota()

    @pl.when(w == 0)
    def _():
        pltpu.sync_copy(counts_hbm, cnt_v)
        c0 = plsc.load_gather(cnt_v, [iota * 8])
        c1 = plsc.load_gather(cnt_v, [(iota + L) * 8])
        acc = jnp.zeros((L,), jnp.int32)
        for e in range(E):
            src = c0 if e < 4 else c1
            s = _sum_scalar(jnp.where(iota // 4 == (e % 4), src, 0))
            acc = acc + s * jnp.where(iota == e, 1, 0)
        tpe_v[...] = acc
        pltpu.sync_copy(tpe_v.at[pl.ds(0, E)], tpe_hbm)

    pltpu.sync_copy(maskf_hbm.at[pl.ds(w * CHUNK, CHUNK)], chunk_v)
    pltpu.sync_copy(probsf_hbm.at[pl.ds(w * CHUNK, CHUNK)], pchunk_v)
    base_flat = w * CHUNK

    def comp(g, off_vec):
        m = chunk_v[pl.ds(g * L, L)]
        mask = m == 1
        incl = _shift_cumsum(m)
        dst = jnp.maximum(off_vec + incl - 1, 0)
        plsc.store_scatter(sel_buf, [dst], base_flat + g * L + iota,
                           mask=mask)
        plsc.store_scatter(pr_buf, [dst], pchunk_v[pl.ds(g * L, L)],
                           mask=mask)
        return off_vec + plsc.all_reduce_population_count(mask)

    lax.fori_loop(0, CHUNK // L, comp, jnp.zeros((L,), jnp.int32), unroll=4)
    pltpu.sync_copy(sel_buf, selpad_hbm.at[pl.ds(base_flat, CHUNK)])
    pltpu.sync_copy(pr_buf, prpad_hbm.at[pl.ds(base_flat, CHUNK)])


@functools.partial(
    pl.kernel,
    out_type=(
        jax.ShapeDtypeStruct((OUT, D), jnp.float32),
        jax.ShapeDtypeStruct((OUT,), jnp.float32),
    ),
    mesh=plsc.VectorSubcoreMesh(**_MESH),
    compiler_params=pltpu.CompilerParams(needs_layout_passes=False),
    scratch_types=[
        pltpu.VMEM((NW * 8,), jnp.int32),
        pltpu.VMEM((NW,), jnp.int32),
        pltpu.VMEM((NW,), jnp.int32),
        pltpu.VMEM((CHUNK,), jnp.int32),
        pltpu.VMEM((CHUNK,), jnp.float32),
        pltpu.VMEM((ROWS_PW,), jnp.int32),
        pltpu.VMEM((ROWS_PW,), jnp.int32),
        pltpu.VMEM((ROWS_PW,), jnp.float32),
        pltpu.VMEM((RG, D), jnp.float32),
        pltpu.VMEM((RG, D), jnp.float32),
        pltpu.SemaphoreType.DMA,
        pltpu.SemaphoreType.DMA,
    ],
)
def _gather_kernel(hidden_hbm, counts_hbm, selpad_hbm, prpad_hbm,
                   out_hbm, pprobs_hbm,
                   cnt_v, pfx_v, ca_v, srow_v, prow_v,
                   sel_v, tok_v, pr_v, rbuf0, rbuf1, sem0, sem1):
    w = _wid()
    iota = _iota()
    row0 = w * ROWS_PW

    pltpu.sync_copy(counts_hbm, cnt_v)
    c0 = plsc.load_gather(cnt_v, [iota * 8])
    c1 = plsc.load_gather(cnt_v, [(iota + L) * 8])
    p0 = _shift_cumsum(c0) - c0
    s0 = _sum_scalar(c0)
    p1 = _shift_cumsum(c1) - c1 + s0
    pfx_v[pl.ds(0, L)] = p0
    pfx_v[pl.ds(L, L)] = p1
    ca_v[pl.ds(0, L)] = c0
    ca_v[pl.ds(L, L)] = c1

    def pull(u, _):
        pu = plsc.load_gather(pfx_v, [_splat(u)])[0]
        cu = plsc.load_gather(ca_v, [_splat(u)])[0]
        a = jnp.maximum(pu, row0)
        b = jnp.minimum(pu + cu, row0 + ROWS_PW)
        n = b - a

        @pl.when(n > 0)
        def _():
            pltpu.sync_copy(selpad_hbm.at[pl.ds(u * CHUNK, CHUNK)], srow_v)
            pltpu.sync_copy(prpad_hbm.at[pl.ds(u * CHUNK, CHUNK)], prow_v)

            def cp(g, _):
                off = g * L + iota
                valid = off < n
                src = jnp.minimum(a - pu + off, CHUNK - 1)
                dst = jnp.minimum(a - row0 + off, ROWS_PW - 1)
                plsc.store_scatter(sel_v, [dst],
                                   plsc.load_gather(srow_v, [src]),
                                   mask=valid)
                plsc.store_scatter(pr_v, [dst],
                                   plsc.load_gather(prow_v, [src]),
                                   mask=valid)
                return 0

            lax.fori_loop(0, (n + L - 1) // L, cp, 0)

        return 0

    lax.fori_loop(0, NW, pull, 0)

    def tok(i, _):
        tok_v[pl.ds(i * L, L)] = lax.bitwise_and(sel_v[pl.ds(i * L, L)], T - 1)
        return 0

    lax.fori_loop(0, ROWS_PW // L, tok, 0, unroll=8)
    pltpu.sync_copy(pr_v, pprobs_hbm.at[pl.ds(row0, ROWS_PW)])

    ng = ROWS_PW // RG

    def start(g, buf, sem):
        pltpu.async_copy(hidden_hbm.at[tok_v.at[pl.ds(g * RG, RG)]], buf, sem)

    def wait(buf, sem):
        pltpu.make_async_copy(hidden_hbm.at[tok_v.at[pl.ds(0, RG)]], buf,
                              sem).wait()

    start(0, rbuf0, sem0)

    def rows(i, _):
        g0 = i * 2
        wait(rbuf0, sem0)
        start(g0 + 1, rbuf1, sem1)
        pltpu.sync_copy(rbuf0, out_hbm.at[pl.ds(row0 + g0 * RG, RG)])
        wait(rbuf1, sem1)

        @pl.when(g0 + 2 < ng)
        def _():
            start(g0 + 2, rbuf0, sem0)

        pltpu.sync_copy(rbuf1, out_hbm.at[pl.ds(row0 + (g0 + 1) * RG, RG)])
        return 0

    lax.fori_loop(0, ng // 2, rows, 0)


def kernel(hidden_states, probs, routing_map):
    maskf = routing_map.T.astype(jnp.int32).reshape(-1)
    probsf = probs.T.reshape(-1)
    counts = _count_kernel(maskf)
    selpad, prpad, tokens_per_expert = _compact_kernel(maskf, probsf, counts)
    permuted_tokens, permuted_probs = _gather_kernel(
        hidden_states, counts, selpad, prpad)
    return permuted_tokens, tokens_per_expert, permuted_probs
